# baseline (speedup 1.0000x reference)
import functools

import jax
import jax.numpy as jnp
from jax import lax
from jax.experimental import pallas as pl
from jax.experimental.pallas import tpu as pltpu
from jax.experimental.pallas import tpu_sc as plsc

N = 10000
E = 320000
D = 128
H = 512
OUT = 256
G = 64

NC = 2
NS = 16
NW = NC * NS
EPW = E // NW
NP = 10112
RPT = NP // NS

R = 400
NB = N // R
BN_INV = float(1.0 / (1.0 + 1e-5) ** 0.5)
NEG = float("-inf")


S = 5
EC = 64
REC = 16
SEG = S * EC
ROUNDS = (EPW - S * REC) // SEG


def _zero_rows(zsrc, acc, sid):
    for r in range(RPT // EC):
        pltpu.sync_copy(zsrc, acc.at[pl.ds(sid * RPT + r * EC, EC), :])
    rem = RPT - (RPT // EC) * EC
    if rem:
        pltpu.sync_copy(zsrc.at[pl.ds(0, rem), :],
                        acc.at[pl.ds(sid * RPT + (RPT // EC) * EC, rem), :])


def _edge_pass(ei, src, acc, rb2, cb2, ci2, ci3, gb, gsem, ssem, isem,
               base, off, gather):
    def gsrc(s, n):
        if not gather:
            return gb.at[pl.ds(0, n), :]
        return gb.at[s, pl.ds(0, n), :] if n != EC else gb.at[s]

    def fire_scat(s, n, cref):
        pltpu.async_copy(gsrc(s, n), acc.at[cref.at[s]], ssem, add=True)

    def drain_scat(s, n, cref):
        pltpu.make_async_copy(gsrc(s, n), acc.at[cref.at[s]], ssem).wait()

    def fire_idx(i, po):
        if gather:
            pltpu.async_copy(ei.at[pl.ds(base + i * SEG, SEG)],
                             rb2.at[pl.ds(po, SEG)], isem)
        pltpu.async_copy(ei.at[pl.ds(E + base + i * SEG, SEG)],
                         cb2.at[pl.ds(po, SEG)], isem)

    def drain_idx(i, po):
        if gather:
            pltpu.make_async_copy(ei.at[pl.ds(base + i * SEG, SEG)],
                                  rb2.at[pl.ds(po, SEG)], isem).wait()
        pltpu.make_async_copy(ei.at[pl.ds(E + base + i * SEG, SEG)],
                              cb2.at[pl.ds(po, SEG)], isem).wait()

    fire_idx(0, 0)

    def round_body(i, c):
        po = lax.rem(i, 2) * SEG
        drain_idx(i, po)

        @pl.when(i > 0)
        def _():
            for s in range(S):
                drain_scat(s, EC, ci2)

        for s in range(S):
            for j in range(EC // 16):
                sl = pl.ds(po + s * EC + j * 16, 16)
                ci2[s, pl.ds(j * 16, 16)] = cb2[sl]
                if gather and off:
                    rb2[sl] = rb2[sl] + off
        if gather:
            for s in range(S):
                pltpu.async_copy(src.at[rb2.at[pl.ds(po + s * EC, EC)]],
                                 gb.at[s], gsem)

        @pl.when(i < ROUNDS - 1)
        def _():
            fire_idx(i + 1, SEG - po)

        if gather:
            for s in range(S):
                pltpu.make_async_copy(src.at[rb2.at[pl.ds(po + s * EC, EC)]],
                                      gb.at[s], gsem).wait()
        for s in range(S):
            fire_scat(s, EC, ci2)
        return c

    lax.fori_loop(0, ROUNDS, round_body, 0)
    for s in range(S):
        drain_scat(s, EC, ci2)

    rem_base = base + ROUNDS * SEG
    if gather:
        pltpu.sync_copy(ei.at[pl.ds(rem_base, S * REC)],
                        rb2.at[pl.ds(0, S * REC)])
    pltpu.sync_copy(ei.at[pl.ds(E + rem_base, S * REC)],
                    cb2.at[pl.ds(0, S * REC)])
    for s in range(S):
        ci3[s, :] = cb2[pl.ds(s * REC, 16)]
        if gather and off:
            rb2[pl.ds(s * REC, 16)] = rb2[pl.ds(s * REC, 16)] + off
    if gather:
        for s in range(S):
            pltpu.async_copy(src.at[rb2.at[pl.ds(s * REC, REC)]],
                             gb.at[s, pl.ds(0, REC), :], gsem)
        for s in range(S):
            pltpu.make_async_copy(src.at[rb2.at[pl.ds(s * REC, REC)]],
                                  gb.at[s, pl.ds(0, REC), :], gsem).wait()
    for s in range(S):
        fire_scat(s, REC, ci3)
    for s in range(S):
        drain_scat(s, REC, ci3)


def _deg_body(ei, out, ones_b, zb, rb2, cb2, ci2, ci3, acc, gsem, ssem, isem):
    cid = lax.axis_index("c")
    sid = lax.axis_index("s")
    wid = sid * NC + cid
    base = wid * EPW

    def fill(i, c):
        for j in range(D // 16):
            ones_b[i, pl.ds(j * 16, 16)] = jnp.ones((16,), jnp.float32)
            zb[i, pl.ds(j * 16, 16)] = jnp.zeros((16,), jnp.float32)
        return c

    lax.fori_loop(0, EC, fill, 0)
    _zero_rows(zb, acc, sid)
    plsc.subcore_barrier()
    _edge_pass(ei, None, acc, rb2, cb2, ci2, ci3, ones_b, gsem, ssem, isem,
               base, 0, gather=False)
    plsc.subcore_barrier()
    pltpu.sync_copy(acc.at[pl.ds(sid * RPT, RPT), :],
                    out.at[cid, pl.ds(sid * RPT, RPT), :])


def _deg_call(ei):
    mesh = plsc.VectorSubcoreMesh(core_axis_name="c", subcore_axis_name="s",
                                  num_cores=NC, num_subcores=NS)
    f = functools.partial(
        pl.kernel,
        out_type=jax.ShapeDtypeStruct((NC, NP, D), jnp.float32),
        mesh=mesh,
        scratch_types=[
            pltpu.VMEM((EC, D), jnp.float32),
            pltpu.VMEM((EC, D), jnp.float32),
            pltpu.VMEM((2 * SEG,), jnp.int32),
            pltpu.VMEM((2 * SEG,), jnp.int32),
            pltpu.VMEM((S, EC), jnp.int32),
            pltpu.VMEM((S, REC), jnp.int32),
            pltpu.VMEM_SHARED((NP, D), jnp.float32),
            pltpu.SemaphoreType.DMA,
            pltpu.SemaphoreType.DMA,
            pltpu.SemaphoreType.DMA,
        ],
    )(_deg_body)
    return f(ei)


def _agg_body(ch, src, ei, out, rb2, cb2, ci2, ci3, gb, acc, gsem, ssem, isem):
    cid = lax.axis_index("c")
    sid = lax.axis_index("s")
    wid = sid * NC + cid
    base = wid * EPW

    for cc in range(ch):
        def fillz(i, c):
            for j in range(D // 16):
                gb[0, i, pl.ds(j * 16, 16)] = jnp.zeros((16,), jnp.float32)
            return c

        lax.fori_loop(0, EC, fillz, 0)
        _zero_rows(gb.at[0], acc, sid)
        plsc.subcore_barrier()
        _edge_pass(ei, src, acc, rb2, cb2, ci2, ci3, gb, gsem, ssem, isem,
                   base, cc * N, gather=True)
        plsc.subcore_barrier()
        pltpu.sync_copy(acc.at[pl.ds(sid * RPT, RPT), :],
                        out.at[cid, pl.ds(cc * NP + sid * RPT, RPT), :])
        plsc.subcore_barrier()


def _agg_call(src, ei, ch):
    mesh = plsc.VectorSubcoreMesh(core_axis_name="c", subcore_axis_name="s",
                                  num_cores=NC, num_subcores=NS)
    f = functools.partial(
        pl.kernel,
        out_type=jax.ShapeDtypeStruct((NC, ch * NP, D), jnp.float32),
        mesh=mesh,
        scratch_types=[
            pltpu.VMEM((2 * SEG,), jnp.int32),
            pltpu.VMEM((2 * SEG,), jnp.int32),
            pltpu.VMEM((S, EC), jnp.int32),
            pltpu.VMEM((S, REC), jnp.int32),
            pltpu.VMEM((S, EC, D), jnp.float32),
            pltpu.VMEM_SHARED((NP, D), jnp.float32),
            pltpu.SemaphoreType.DMA,
            pltpu.SemaphoreType.DMA,
            pltpu.SemaphoreType.DMA,
        ],
    )(functools.partial(_agg_body, ch))
    return f(src, ei)


def _t1_body(dp_ref, x_ref, dis_ref, xs_ref):
    deg = dp_ref[0, :, 0:1] + dp_ref[1, :, 0:1] + 2.0
    dis = lax.rsqrt(deg)
    dis_ref[...] = dis
    xs_ref[...] = dis * x_ref[...]


def _t1_call(dp, x):
    return pl.pallas_call(
        _t1_body,
        grid=(NB,),
        in_specs=[
            pl.BlockSpec((NC, R, D), lambda i: (0, i, 0)),
            pl.BlockSpec((R, D), lambda i: (i, 0)),
        ],
        out_specs=[
            pl.BlockSpec((R, 1), lambda i: (i, 0)),
            pl.BlockSpec((R, D), lambda i: (i, 0)),
        ],
        out_shape=[
            jax.ShapeDtypeStruct((N, 1), jnp.float32),
            jax.ShapeDtypeStruct((N, D), jnp.float32),
        ],
    )(dp, x)


def _t2_body(a1_ref, xs_ref, dis_ref, w1_ref, b1_ref, g1_ref, be1_ref, hs_ref):
    dis = dis_ref[...]
    p = a1_ref[0] + a1_ref[1]
    agg = dis * (p + 2.0 * xs_ref[...])
    z = jnp.dot(agg, w1_ref[...], preferred_element_type=jnp.float32)
    z = (z + b1_ref[...]) * (g1_ref[...] * BN_INV) + be1_ref[...]
    h = jnp.where(z >= 0, z, 0.01 * z)
    hs = dis * h
    for c in range(H // D):
        hs_ref[c] = hs[:, c * D:(c + 1) * D]


def _t2_call(a1, xs1, dis, w1, b1, g1, be1):
    return pl.pallas_call(
        _t2_body,
        grid=(NB,),
        in_specs=[
            pl.BlockSpec((NC, R, D), lambda i: (0, i, 0)),
            pl.BlockSpec((R, D), lambda i: (i, 0)),
            pl.BlockSpec((R, 1), lambda i: (i, 0)),
            pl.BlockSpec((D, H), lambda i: (0, 0)),
            pl.BlockSpec((1, H), lambda i: (0, 0)),
            pl.BlockSpec((1, H), lambda i: (0, 0)),
            pl.BlockSpec((1, H), lambda i: (0, 0)),
        ],
        out_specs=pl.BlockSpec((H // D, R, D), lambda i: (0, i, 0)),
        out_shape=jax.ShapeDtypeStruct((H // D, N, D), jnp.float32),
    )(a1, xs1, dis, w1, b1, g1, be1)


def _t3_body(a2_ref, hs_ref, dis_ref, w2_ref, b2_ref, g2_ref, be2_ref,
             bt_ref, wf_ref, bf_ref, out_ref, zacc, pacc):
    i = pl.program_id(0)
    c = pl.program_id(1)

    @pl.when(jnp.logical_and(i == 0, c == 0))
    def _():
        pacc[...] = jnp.full((G, H), NEG, jnp.float32)

    @pl.when(c == 0)
    def _():
        zacc[...] = jnp.zeros((R, H), jnp.float32)

    dis = dis_ref[...]
    p = a2_ref[0, 0] + a2_ref[1, 0]
    agg = dis * (p + 2.0 * hs_ref[0])
    zacc[...] += jnp.dot(agg, w2_ref[0], preferred_element_type=jnp.float32)

    @pl.when(c == H // D - 1)
    def _():
        z = (zacc[...] + b2_ref[...]) * (g2_ref[...] * BN_INV) + be2_ref[...]
        h2 = jnp.where(z >= 0, z, 0.01 * z)
        bb = bt_ref[...]
        gid = lax.broadcasted_iota(jnp.int32, (G, 1), 0)

        def seg(g, carry):
            m = bb == g
            v = jnp.max(jnp.where(m, h2, NEG), axis=0, keepdims=True)
            upd = jnp.maximum(pacc[...], v)
            pacc[...] = jnp.where(gid == g, upd, pacc[...])
            return carry

        lax.fori_loop(jnp.min(bb), jnp.max(bb) + 1, seg, 0)

        @pl.when(i == NB - 1)
        def _():
            pooled = pacc[...]
            pooled = jnp.where(pooled == NEG, 0.0, pooled)
            out_ref[...] = (jnp.dot(pooled, wf_ref[...],
                                    preferred_element_type=jnp.float32)
                            + bf_ref[...])


def _t3_call(a2, hs4, dis, w2, b2, g2, be2, bt, wf, bf):
    nch = H // D
    return pl.pallas_call(
        _t3_body,
        grid=(NB, nch),
        in_specs=[
            pl.BlockSpec((NC, 1, R, D), lambda i, c: (0, c, i, 0)),
            pl.BlockSpec((1, R, D), lambda i, c: (c, i, 0)),
            pl.BlockSpec((R, 1), lambda i, c: (i, 0)),
            pl.BlockSpec((1, D, H), lambda i, c: (c, 0, 0)),
            pl.BlockSpec((1, H), lambda i, c: (0, 0)),
            pl.BlockSpec((1, H), lambda i, c: (0, 0)),
            pl.BlockSpec((1, H), lambda i, c: (0, 0)),
            pl.BlockSpec((R, 1), lambda i, c: (i, 0)),
            pl.BlockSpec((H, OUT), lambda i, c: (0, 0)),
            pl.BlockSpec((1, OUT), lambda i, c: (0, 0)),
        ],
        out_specs=pl.BlockSpec((G, OUT), lambda i, c: (0, 0)),
        out_shape=jax.ShapeDtypeStruct((G, OUT), jnp.float32),
        scratch_shapes=[
            pltpu.VMEM((R, H), jnp.float32),
            pltpu.VMEM((G, H), jnp.float32),
        ],
    )(a2, hs4, dis, w2, b2, g2, be2, bt, wf, bf)


def kernel(x, edge_index, batch, W1, b1, gamma1, beta1,
           W2, b2, gamma2, beta2, Wf, bf):
    ei = edge_index.astype(jnp.int32).reshape(2 * E)
    degp = _deg_call(ei)
    dis, xs1 = _t1_call(degp, x)
    a1 = _agg_call(xs1, ei, 1)
    hs4 = _t2_call(a1, xs1, dis, W1,
                   b1.reshape(1, H), gamma1.reshape(1, H), beta1.reshape(1, H))
    a2 = _agg_call(hs4.reshape(H // D * N, D), ei, H // D)
    out = _t3_call(a2.reshape(NC, H // D, NP, D), hs4, dis,
                   W2.reshape(H // D, D, H),
                   b2.reshape(1, H), gamma2.reshape(1, H), beta2.reshape(1, H),
                   batch.reshape(N, 1).astype(jnp.int32),
                   Wf, bf.reshape(1, OUT))
    return out


# speedup vs baseline: 1.1602x; 1.1602x over previous
import functools

import jax
import jax.numpy as jnp
from jax import lax
from jax.experimental import pallas as pl
from jax.experimental.pallas import tpu as pltpu
from jax.experimental.pallas import tpu_sc as plsc

N = 10000
E = 320000
D = 128
H = 512
OUT = 256
G = 64

NC = 2
NS = 16
NW = NC * NS
EPW = E // NW
NP = 10112
RPT = NP // NS

R = 400
NB = N // R
BN_INV = float(1.0 / (1.0 + 1e-5) ** 0.5)
NEG = float("-inf")


S = 5
EC = 32
REC = 16
SEG = S * EC
ROUNDS = (EPW - S * REC) // SEG
RPAIRS = ROUNDS // 2


def _zero_rows(zsrc, acc, sid):
    for r in range(RPT // EC):
        pltpu.sync_copy(zsrc, acc.at[pl.ds(sid * RPT + r * EC, EC), :])
    rem = RPT - (RPT // EC) * EC
    if rem:
        pltpu.sync_copy(zsrc.at[pl.ds(0, rem), :],
                        acc.at[pl.ds(sid * RPT + (RPT // EC) * EC, rem), :])


def _edge_pass(ei, src, acc, rb2, cb2, ci2, ci3, gb, gsem, ssem, isem,
               base, off, gather):
    def gsrc(pidx, s, n):
        if not gather:
            return gb.at[pl.ds(0, n), :]
        return gb.at[pidx + s, pl.ds(0, n), :] if n != EC else gb.at[pidx + s]

    def fire_scat(pidx, s, n, cref):
        pltpu.async_copy(gsrc(pidx, s, n), acc.at[cref.at[pidx + s]],
                         ssem, add=True)

    def drain_scat(pidx, s, n, cref):
        pltpu.make_async_copy(gsrc(pidx, s, n), acc.at[cref.at[pidx + s]],
                              ssem).wait()

    def fire_idx(r, po):
        if gather:
            pltpu.async_copy(ei.at[pl.ds(base + r * SEG, SEG)],
                             rb2.at[pl.ds(po, SEG)], isem)
        pltpu.async_copy(ei.at[pl.ds(E + base + r * SEG, SEG)],
                         cb2.at[pl.ds(po, SEG)], isem)

    def drain_idx(r, po):
        if gather:
            pltpu.make_async_copy(ei.at[pl.ds(base + r * SEG, SEG)],
                                  rb2.at[pl.ds(po, SEG)], isem).wait()
        pltpu.make_async_copy(ei.at[pl.ds(E + base + r * SEG, SEG)],
                              cb2.at[pl.ds(po, SEG)], isem).wait()

    fire_idx(0, 0)
    fire_idx(1, SEG)

    def pair_body(i, c):
        for p in (0, 1):
            r = 2 * i + p
            pidx = p * S
            po = p * SEG
            drain_idx(r, po)

            @pl.when(i > 0)
            def _():
                for s in range(S):
                    drain_scat(pidx, s, EC, ci2)

            for s in range(S):
                for j in range(EC // 16):
                    sl = pl.ds(po + s * EC + j * 16, 16)
                    ci2[pidx + s, pl.ds(j * 16, 16)] = cb2[sl]
                    if gather and off:
                        rb2[sl] = rb2[sl] + off
            if gather:
                for s in range(S):
                    pltpu.async_copy(src.at[rb2.at[pl.ds(po + s * EC, EC)]],
                                     gb.at[pidx + s], gsem)
                for s in range(S):
                    pltpu.make_async_copy(
                        src.at[rb2.at[pl.ds(po + s * EC, EC)]],
                        gb.at[pidx + s], gsem).wait()
            for s in range(S):
                fire_scat(pidx, s, EC, ci2)

            @pl.when(i < RPAIRS - 1)
            def _():
                fire_idx(r + 2, po)
        return c

    lax.fori_loop(0, RPAIRS, pair_body, 0)
    for p in (0, 1):
        for s in range(S):
            drain_scat(p * S, s, EC, ci2)

    rem_base = base + ROUNDS * SEG
    if gather:
        pltpu.sync_copy(ei.at[pl.ds(rem_base, S * REC)],
                        rb2.at[pl.ds(0, S * REC)])
    pltpu.sync_copy(ei.at[pl.ds(E + rem_base, S * REC)],
                    cb2.at[pl.ds(0, S * REC)])
    for s in range(S):
        ci3[s, :] = cb2[pl.ds(s * REC, 16)]
        if gather and off:
            rb2[pl.ds(s * REC, 16)] = rb2[pl.ds(s * REC, 16)] + off
    if gather:
        for s in range(S):
            pltpu.async_copy(src.at[rb2.at[pl.ds(s * REC, REC)]],
                             gb.at[s, pl.ds(0, REC), :], gsem)
        for s in range(S):
            pltpu.make_async_copy(src.at[rb2.at[pl.ds(s * REC, REC)]],
                                  gb.at[s, pl.ds(0, REC), :], gsem).wait()
    for s in range(S):
        pltpu.async_copy(gb.at[s, pl.ds(0, REC), :] if gather
                         else gb.at[pl.ds(0, REC), :],
                         acc.at[ci3.at[s]], ssem, add=True)
    for s in range(S):
        pltpu.make_async_copy(gb.at[s, pl.ds(0, REC), :] if gather
                              else gb.at[pl.ds(0, REC), :],
                              acc.at[ci3.at[s]], ssem).wait()


def _deg_body(ei, out, ones_b, zb, rb2, cb2, ci2, ci3, acc, gsem, ssem, isem):
    cid = lax.axis_index("c")
    sid = lax.axis_index("s")
    wid = sid * NC + cid
    base = wid * EPW

    def fill(i, c):
        for j in range(D // 16):
            ones_b[i, pl.ds(j * 16, 16)] = jnp.ones((16,), jnp.float32)
            zb[i, pl.ds(j * 16, 16)] = jnp.zeros((16,), jnp.float32)
        return c

    lax.fori_loop(0, EC, fill, 0)
    _zero_rows(zb, acc, sid)
    plsc.subcore_barrier()
    _edge_pass(ei, None, acc, rb2, cb2, ci2, ci3, ones_b, gsem, ssem, isem,
               base, 0, gather=False)
    plsc.subcore_barrier()
    pltpu.sync_copy(acc.at[pl.ds(sid * RPT, RPT), :],
                    out.at[cid, pl.ds(sid * RPT, RPT), :])


def _deg_call(ei):
    mesh = plsc.VectorSubcoreMesh(core_axis_name="c", subcore_axis_name="s",
                                  num_cores=NC, num_subcores=NS)
    f = functools.partial(
        pl.kernel,
        out_type=jax.ShapeDtypeStruct((NC, NP, D), jnp.float32),
        mesh=mesh,
        scratch_types=[
            pltpu.VMEM((EC, D), jnp.float32),
            pltpu.VMEM((EC, D), jnp.float32),
            pltpu.VMEM((2 * SEG,), jnp.int32),
            pltpu.VMEM((2 * SEG,), jnp.int32),
            pltpu.VMEM((2 * S, EC), jnp.int32),
            pltpu.VMEM((S, REC), jnp.int32),
            pltpu.VMEM_SHARED((NP, D), jnp.float32),
            pltpu.SemaphoreType.DMA,
            pltpu.SemaphoreType.DMA,
            pltpu.SemaphoreType.DMA,
        ],
    )(_deg_body)
    return f(ei)


def _agg_body(ch, src, ei, out, rb2, cb2, ci2, ci3, gb, acc, gsem, ssem, isem):
    cid = lax.axis_index("c")
    sid = lax.axis_index("s")
    wid = sid * NC + cid
    base = wid * EPW

    for cc in range(ch):
        def fillz(i, c):
            for j in range(D // 16):
                gb[0, i, pl.ds(j * 16, 16)] = jnp.zeros((16,), jnp.float32)
            return c

        lax.fori_loop(0, EC, fillz, 0)
        _zero_rows(gb.at[0], acc, sid)
        plsc.subcore_barrier()
        _edge_pass(ei, src, acc, rb2, cb2, ci2, ci3, gb, gsem, ssem, isem,
                   base, cc * N, gather=True)
        plsc.subcore_barrier()
        pltpu.sync_copy(acc.at[pl.ds(sid * RPT, RPT), :],
                        out.at[cid, pl.ds(cc * NP + sid * RPT, RPT), :])
        plsc.subcore_barrier()


def _agg_call(src, ei, ch):
    mesh = plsc.VectorSubcoreMesh(core_axis_name="c", subcore_axis_name="s",
                                  num_cores=NC, num_subcores=NS)
    f = functools.partial(
        pl.kernel,
        out_type=jax.ShapeDtypeStruct((NC, ch * NP, D), jnp.float32),
        mesh=mesh,
        scratch_types=[
            pltpu.VMEM((2 * SEG,), jnp.int32),
            pltpu.VMEM((2 * SEG,), jnp.int32),
            pltpu.VMEM((2 * S, EC), jnp.int32),
            pltpu.VMEM((S, REC), jnp.int32),
            pltpu.VMEM((2 * S, EC, D), jnp.float32),
            pltpu.VMEM_SHARED((NP, D), jnp.float32),
            pltpu.SemaphoreType.DMA,
            pltpu.SemaphoreType.DMA,
            pltpu.SemaphoreType.DMA,
        ],
    )(functools.partial(_agg_body, ch))
    return f(src, ei)


def _t1_body(dp_ref, x_ref, dis_ref, xs_ref):
    deg = dp_ref[0, :, 0:1] + dp_ref[1, :, 0:1] + 2.0
    dis = lax.rsqrt(deg)
    dis_ref[...] = dis
    xs_ref[...] = dis * x_ref[...]


def _t1_call(dp, x):
    return pl.pallas_call(
        _t1_body,
        grid=(NB,),
        in_specs=[
            pl.BlockSpec((NC, R, D), lambda i: (0, i, 0)),
            pl.BlockSpec((R, D), lambda i: (i, 0)),
        ],
        out_specs=[
            pl.BlockSpec((R, 1), lambda i: (i, 0)),
            pl.BlockSpec((R, D), lambda i: (i, 0)),
        ],
        out_shape=[
            jax.ShapeDtypeStruct((N, 1), jnp.float32),
            jax.ShapeDtypeStruct((N, D), jnp.float32),
        ],
    )(dp, x)


def _t2_body(a1_ref, xs_ref, dis_ref, w1_ref, b1_ref, g1_ref, be1_ref, hs_ref):
    dis = dis_ref[...]
    p = a1_ref[0] + a1_ref[1]
    agg = dis * (p + 2.0 * xs_ref[...])
    z = jnp.dot(agg, w1_ref[...], preferred_element_type=jnp.float32)
    z = (z + b1_ref[...]) * (g1_ref[...] * BN_INV) + be1_ref[...]
    h = jnp.where(z >= 0, z, 0.01 * z)
    hs = dis * h
    for c in range(H // D):
        hs_ref[c] = hs[:, c * D:(c + 1) * D]


def _t2_call(a1, xs1, dis, w1, b1, g1, be1):
    return pl.pallas_call(
        _t2_body,
        grid=(NB,),
        in_specs=[
            pl.BlockSpec((NC, R, D), lambda i: (0, i, 0)),
            pl.BlockSpec((R, D), lambda i: (i, 0)),
            pl.BlockSpec((R, 1), lambda i: (i, 0)),
            pl.BlockSpec((D, H), lambda i: (0, 0)),
            pl.BlockSpec((1, H), lambda i: (0, 0)),
            pl.BlockSpec((1, H), lambda i: (0, 0)),
            pl.BlockSpec((1, H), lambda i: (0, 0)),
        ],
        out_specs=pl.BlockSpec((H // D, R, D), lambda i: (0, i, 0)),
        out_shape=jax.ShapeDtypeStruct((H // D, N, D), jnp.float32),
    )(a1, xs1, dis, w1, b1, g1, be1)


def _t3_body(a2_ref, hs_ref, dis_ref, w2_ref, b2_ref, g2_ref, be2_ref,
             bt_ref, wf_ref, bf_ref, out_ref, zacc, pacc):
    i = pl.program_id(0)
    c = pl.program_id(1)

    @pl.when(jnp.logical_and(i == 0, c == 0))
    def _():
        pacc[...] = jnp.full((G, H), NEG, jnp.float32)

    @pl.when(c == 0)
    def _():
        zacc[...] = jnp.zeros((R, H), jnp.float32)

    dis = dis_ref[...]
    p = a2_ref[0, 0] + a2_ref[1, 0]
    agg = dis * (p + 2.0 * hs_ref[0])
    zacc[...] += jnp.dot(agg, w2_ref[0], preferred_element_type=jnp.float32)

    @pl.when(c == H // D - 1)
    def _():
        z = (zacc[...] + b2_ref[...]) * (g2_ref[...] * BN_INV) + be2_ref[...]
        h2 = jnp.where(z >= 0, z, 0.01 * z)
        bb = bt_ref[...]
        gid = lax.broadcasted_iota(jnp.int32, (G, 1), 0)

        def seg(g, carry):
            m = bb == g
            v = jnp.max(jnp.where(m, h2, NEG), axis=0, keepdims=True)
            upd = jnp.maximum(pacc[...], v)
            pacc[...] = jnp.where(gid == g, upd, pacc[...])
            return carry

        lax.fori_loop(jnp.min(bb), jnp.max(bb) + 1, seg, 0)

        @pl.when(i == NB - 1)
        def _():
            pooled = pacc[...]
            pooled = jnp.where(pooled == NEG, 0.0, pooled)
            out_ref[...] = (jnp.dot(pooled, wf_ref[...],
                                    preferred_element_type=jnp.float32)
                            + bf_ref[...])


def _t3_call(a2, hs4, dis, w2, b2, g2, be2, bt, wf, bf):
    nch = H // D
    return pl.pallas_call(
        _t3_body,
        grid=(NB, nch),
        in_specs=[
            pl.BlockSpec((NC, 1, R, D), lambda i, c: (0, c, i, 0)),
            pl.BlockSpec((1, R, D), lambda i, c: (c, i, 0)),
            pl.BlockSpec((R, 1), lambda i, c: (i, 0)),
            pl.BlockSpec((1, D, H), lambda i, c: (c, 0, 0)),
            pl.BlockSpec((1, H), lambda i, c: (0, 0)),
            pl.BlockSpec((1, H), lambda i, c: (0, 0)),
            pl.BlockSpec((1, H), lambda i, c: (0, 0)),
            pl.BlockSpec((R, 1), lambda i, c: (i, 0)),
            pl.BlockSpec((H, OUT), lambda i, c: (0, 0)),
            pl.BlockSpec((1, OUT), lambda i, c: (0, 0)),
        ],
        out_specs=pl.BlockSpec((G, OUT), lambda i, c: (0, 0)),
        out_shape=jax.ShapeDtypeStruct((G, OUT), jnp.float32),
        scratch_shapes=[
            pltpu.VMEM((R, H), jnp.float32),
            pltpu.VMEM((G, H), jnp.float32),
        ],
    )(a2, hs4, dis, w2, b2, g2, be2, bt, wf, bf)


def kernel(x, edge_index, batch, W1, b1, gamma1, beta1,
           W2, b2, gamma2, beta2, Wf, bf):
    ei = edge_index.astype(jnp.int32).reshape(2 * E)
    degp = _deg_call(ei)
    dis, xs1 = _t1_call(degp, x)
    a1 = _agg_call(xs1, ei, 1)
    hs4 = _t2_call(a1, xs1, dis, W1,
                   b1.reshape(1, H), gamma1.reshape(1, H), beta1.reshape(1, H))
    a2 = _agg_call(hs4.reshape(H // D * N, D), ei, H // D)
    out = _t3_call(a2.reshape(NC, H // D, NP, D), hs4, dis,
                   W2.reshape(H // D, D, H),
                   b2.reshape(1, H), gamma2.reshape(1, H), beta2.reshape(1, H),
                   batch.reshape(N, 1).astype(jnp.int32),
                   Wf, bf.reshape(1, OUT))
    return out
